# baseline (speedup 1.0000x reference)
import functools

import numpy as np
import jax
import jax.numpy as jnp
from jax import lax
from jax.experimental import pallas as pl
from jax.experimental.pallas import tpu as pltpu
from jax.experimental.pallas import tpu_sc as plsc

_C = 1000
_CP = 1024
_B = 16384
_K = 20
_NP = _K * (_K - 1) // 2
_W = 64

_BLK = 2048

_NC = 2
_NS = 16
_L = 16
_RH = _CP // _NC
_TW = _RH * _CP
_CHUNK = _B // _NS
_RT = _RH // _NS
_TSB = _RT * _CP
_ZB = 8192
_CC = _CP // _NS


def _argmax_body(s_ref, out_ref):
    s = s_ref[...]
    m = jnp.max(s, axis=1, keepdims=True)
    col = lax.broadcasted_iota(jnp.int32, s.shape, 1)
    out_ref[...] = jnp.min(jnp.where(s == m, col, _C), axis=1).astype(jnp.int32)


def _row_argmax(scores):
    return pl.pallas_call(
        _argmax_body,
        grid=(_B // _BLK,),
        in_specs=[pl.BlockSpec((_BLK, _C), lambda i: (i, 0))],
        out_specs=pl.BlockSpec((_BLK,), lambda i: (i,)),
        out_shape=jax.ShapeDtypeStruct((_B,), jnp.int32),
        compiler_params=pltpu.CompilerParams(allow_input_fusion=[True]),
    )(scores)


def _hist_body(y_hbm, p_hbm, entry_hbm, argr_hbm,
               yv, pv, idxv, valv, zbuf, tsb, maxb, argb, mt, at, fm, fa,
               conf_sh, maxp_sh, argp_sh):
    c = lax.axis_index("c")
    s = lax.axis_index("s")
    base = s * _CHUNK
    row0 = _RH * c + _RT * s

    def zstep(i, _):
        zbuf[pl.ds(i * _L, _L)] = jnp.zeros((_L,), jnp.float32)
        return 0
    lax.fori_loop(0, _ZB // _L, zstep, 0)
    for q in range(_TSB // _ZB):
        pltpu.sync_copy(zbuf, conf_sh.at[pl.ds(s * _TSB + q * _ZB, _ZB)])

    for k in range(_CHUNK // 128):
        pltpu.sync_copy(y_hbm.at[pl.ds(base + k * 128, 128)], yv.at[k])
        pltpu.sync_copy(p_hbm.at[pl.ds(base + k * 128, 128)], pv.at[k])
    cbase = c * _RH * _CP
    for k in range(_CHUNK // 128):
        for i in range(128 // _L):
            yi = yv[k, pl.ds(i * _L, _L)]
            pi = pv[k, pl.ds(i * _L, _L)]
            code = yi * _CP + pi - cbase
            ok = (code >= 0) & (code < _TW) & (yi != pi)
            dummy = (lax.broadcasted_iota(jnp.int32, (_L,), 0)
                     + base + k * 128 + i * _L)
            idxv[k, pl.ds(i * _L, _L)] = jnp.where(ok, code, dummy)
            valv[k, pl.ds(i * _L, _L)] = jnp.where(
                ok, jnp.full((_L,), 1.0, jnp.float32),
                jnp.zeros((_L,), jnp.float32))

    plsc.subcore_barrier()
    for k in range(_CHUNK // 128):
        pltpu.sync_copy(valv.at[k], conf_sh.at[idxv.at[k]], add=True)
    plsc.subcore_barrier()

    pltpu.sync_copy(conf_sh.at[pl.ds(s * _TSB, _TSB)], tsb)

    def col_chunk(v, _):
        def row_step(r, carry):
            m, a = carry
            for u in range(4):
                x = tsb[pl.ds((r * 4 + u) * _CP + v * _L, _L)]
                upd = x > m
                aabs = jnp.full((_L,), 0, jnp.int32) + (row0 + r * 4 + u)
                m = jnp.where(upd, x, m)
                a = jnp.where(upd, aabs, a)
            return m, a
        m, a = lax.fori_loop(
            0, _RT // 4, row_step,
            (jnp.full((_L,), -1.0, jnp.float32), jnp.zeros((_L,), jnp.int32)))
        maxb[pl.ds(v * _L, _L)] = m
        argb[pl.ds(v * _L, _L)] = a
        return 0
    lax.fori_loop(0, _CP // _L, col_chunk, 0)

    pltpu.sync_copy(maxb, maxp_sh.at[pl.ds(s * _CP, _CP)])
    pltpu.sync_copy(argb, argp_sh.at[pl.ds(s * _CP, _CP)])
    plsc.subcore_barrier()

    for t in range(_NS):
        pltpu.sync_copy(maxp_sh.at[pl.ds(t * _CP + s * _CC, _CC)], mt.at[t])
        pltpu.sync_copy(argp_sh.at[pl.ds(t * _CP + s * _CC, _CC)], at.at[t])
    for q in range(_CC // _L):
        def comb_step(t, carry):
            m, a = carry
            x = mt[t, pl.ds(q * _L, _L)]
            xa = at[t, pl.ds(q * _L, _L)]
            upd = x > m
            return jnp.where(upd, x, m), jnp.where(upd, xa, a)
        m0 = mt[0, pl.ds(q * _L, _L)]
        a0 = at[0, pl.ds(q * _L, _L)]
        m, a = lax.fori_loop(1, _NS, comb_step, (m0, a0))
        fm[pl.ds(q * _L, _L)] = m
        fa[pl.ds(q * _L, _L)] = a

    pltpu.sync_copy(fm, entry_hbm.at[c, pl.ds(s * _CC, _CC)])
    pltpu.sync_copy(fa, argr_hbm.at[c, pl.ds(s * _CC, _CC)])


def _histogram(y, preds):
    mesh = plsc.VectorSubcoreMesh(core_axis_name="c", subcore_axis_name="s",
                                  num_cores=_NC, num_subcores=_NS)
    f = pl.kernel(
        _hist_body,
        out_type=[jax.ShapeDtypeStruct((_NC, _CP), jnp.float32),
                  jax.ShapeDtypeStruct((_NC, _CP), jnp.int32)],
        mesh=mesh,
        scratch_types=[
            pltpu.VMEM((_CHUNK // 128, 128), jnp.int32),
            pltpu.VMEM((_CHUNK // 128, 128), jnp.int32),
            pltpu.VMEM((_CHUNK // 128, 128), jnp.int32),
            pltpu.VMEM((_CHUNK // 128, 128), jnp.float32),
            pltpu.VMEM((_ZB,), jnp.float32),
            pltpu.VMEM((_TSB,), jnp.float32),
            pltpu.VMEM((_CP,), jnp.float32),
            pltpu.VMEM((_CP,), jnp.int32),
            pltpu.VMEM((_NS, _CC), jnp.float32),
            pltpu.VMEM((_NS, _CC), jnp.int32),
            pltpu.VMEM((_CC,), jnp.float32),
            pltpu.VMEM((_CC,), jnp.int32),
            pltpu.VMEM_SHARED((_TW,), jnp.float32),
            pltpu.VMEM_SHARED((_NS * _CP,), jnp.float32),
            pltpu.VMEM_SHARED((_NS * _CP,), jnp.int32),
        ],
    )
    return f(y, preds)


def _post_body(e_ref, a_ref, et_ref, at_ref, pairs_ref):
    e = e_ref[...]
    a = a_ref[...]
    e0, e1 = e[0:1, :], e[1:2, :]
    a0, a1 = a[0:1, :], a[1:2, :]
    take1 = e1 > e0
    entry = jnp.where(take1, e1, e0)
    idx = jnp.where(take1, a1, a0)
    et = et_ref[...]
    at = at_ref[...]
    e0c, e1c = et[:, 0:1], et[:, 1:2]
    a0c, a1c = at[:, 0:1], at[:, 1:2]
    take1c = e1c > e0c
    entry_c = jnp.where(take1c, e1c, e0c)
    idx_c = jnp.where(take1c, a1c, a0c)

    cio = lax.broadcasted_iota(jnp.int32, (_C, _CP), 0)
    pio = lax.broadcasted_iota(jnp.int32, (_C, _CP), 1)
    m = ((idx == cio) & (pio < _C)).astype(jnp.float32)
    counts = jnp.sum(m, axis=1, keepdims=True)
    totals = jnp.sum(m * entry, axis=1, keepdims=True)
    cioP = lax.broadcasted_iota(jnp.int32, (_CP, _C), 1)
    pioP = lax.broadcasted_iota(jnp.int32, (_CP, _C), 0)
    mP = ((idx_c == cioP) & (pioP < _C)).astype(jnp.float32)
    counts_r = jnp.sum(mP, axis=0, keepdims=True)
    totals_r = jnp.sum(mP * entry_c, axis=0, keepdims=True)

    cidx_c = lax.broadcasted_iota(jnp.int32, (_C, 1), 0)
    t_c = jnp.where(counts > 0.0, totals, -1.0)
    k2_c = (jnp.where(counts > 0.0, counts, -1.0) * 1024.0
            + (1023 - cidx_c).astype(jnp.float32))
    cidx_r = lax.broadcasted_iota(jnp.int32, (1, _C), 1)
    t_r = jnp.where(counts_r > 0.0, totals_r, -1.0)
    k2_r = (jnp.where(counts_r > 0.0, counts_r, -1.0) * 1024.0
            + (1023 - cidx_r).astype(jnp.float32))

    dom = (t_c > t_r) | ((t_c == t_r) & (k2_c > k2_r))
    rank = jnp.sum(dom.astype(jnp.float32), axis=0, keepdims=True)

    kio2 = lax.broadcasted_iota(jnp.int32, (_K, _C), 0)
    cio2 = lax.broadcasted_iota(jnp.int32, (_K, _C), 1)
    r = (kio2.astype(jnp.float32) == rank).astype(jnp.float32)

    ranked = jnp.sum(r * cio2.astype(jnp.float32), axis=1, keepdims=True)

    hit = r[:, :_W]
    tril = (lax.broadcasted_iota(jnp.int32, (_K, _K), 1)
            <= lax.broadcasted_iota(jnp.int32, (_K, _K), 0)).astype(jnp.float32)
    prefix = jnp.dot(tril, hit, preferred_element_type=jnp.float32)
    alive = (prefix == 0.0).astype(jnp.float32)
    incl = (lax.broadcasted_iota(jnp.int32, (_W, _W), 0)
            <= lax.broadcasted_iota(jnp.int32, (_W, _W), 1)).astype(jnp.float32)
    rank = jnp.dot(alive, incl, preferred_element_type=jnp.float32)

    kio = lax.broadcasted_iota(jnp.int32, (_NP, _K), 0)
    iio = lax.broadcasted_iota(jnp.int32, (_NP, _K), 1)
    start = (_K - 1) * iio - (iio * (iio - 1)) // 2
    g1 = ((kio >= start) & (kio < start + (_K - 1) - iio)).astype(jnp.float32)
    j1 = jnp.sum(g1 * (kio - start).astype(jnp.float32), axis=1,
                 keepdims=True) + 1.0

    first = jnp.dot(g1, ranked, preferred_element_type=jnp.float32)
    ai = jnp.dot(g1, alive, preferred_element_type=jnp.float32)
    ri = jnp.dot(g1, rank, preferred_element_type=jnp.float32)
    vio = lax.broadcasted_iota(jnp.int32, (_NP, _W), 1).astype(jnp.float32)
    second = jnp.sum(vio * ai * (ri == j1).astype(jnp.float32), axis=1,
                     keepdims=True)
    pairs_ref[...] = jnp.concatenate([first, second], axis=1).astype(jnp.int32)


def _post(entry2, argr2):
    return pl.pallas_call(
        _post_body,
        in_specs=[pl.BlockSpec((_NC, _CP), lambda: (0, 0)),
                  pl.BlockSpec((_NC, _CP), lambda: (0, 0)),
                  pl.BlockSpec((_CP, _NC), lambda: (0, 0)),
                  pl.BlockSpec((_CP, _NC), lambda: (0, 0))],
        out_specs=pl.BlockSpec((_NP, 2), lambda: (0, 0)),
        out_shape=jax.ShapeDtypeStruct((_NP, 2), jnp.int32),
    )(entry2, argr2, entry2.T, argr2.T)


def kernel(y, scores):
    preds = _row_argmax(scores)
    entry2, argr2 = _histogram(y, preds)
    return _post(entry2, argr2)


# speedup vs baseline: 1.0690x; 1.0690x over previous
import functools

import numpy as np
import jax
import jax.numpy as jnp
from jax import lax
from jax.experimental import pallas as pl
from jax.experimental.pallas import tpu as pltpu
from jax.experimental.pallas import tpu_sc as plsc

_C = 1000
_CP = 1024
_B = 16384
_K = 20
_NP = _K * (_K - 1) // 2
_W = 64

_BLK = 2048

_NC = 2
_NS = 16
_L = 16
_RH = _CP // _NC
_TW = _RH * _CP
_CHUNK = _B // _NS
_RT = _RH // _NS
_TSB = _RT * _CP
_ZB = 8192
_CC = _CP // _NS


def _argmax_body(s_ref, out_ref):
    s = s_ref[...]
    m = jnp.max(s, axis=1, keepdims=True)
    col = lax.broadcasted_iota(jnp.int32, s.shape, 1)
    out_ref[...] = jnp.min(jnp.where(s == m, col, _C), axis=1).astype(jnp.int32)


def _row_argmax(scores):
    return pl.pallas_call(
        _argmax_body,
        grid=(_B // _BLK,),
        in_specs=[pl.BlockSpec((_BLK, _C), lambda i: (i, 0))],
        out_specs=pl.BlockSpec((_BLK,), lambda i: (i,)),
        out_shape=jax.ShapeDtypeStruct((_B,), jnp.int32),
    )(scores)


def _hist_body(y_hbm, p_hbm, entry_hbm, argr_hbm,
               yv, pv, idxv, valv, zbuf, tsb, maxb, argb, mpv, apv, fm, fa,
               conf_sh, maxp_sh, argp_sh):
    c = lax.axis_index("c")
    s = lax.axis_index("s")
    base = s * _CHUNK
    row0 = _RH * c + _RT * s

    def zstep(i, _):
        zbuf[pl.ds(i * _L, _L)] = jnp.zeros((_L,), jnp.float32)
        return 0
    lax.fori_loop(0, _ZB // _L, zstep, 0)
    for q in range(_TSB // _ZB):
        pltpu.sync_copy(zbuf, conf_sh.at[pl.ds(s * _TSB + q * _ZB, _ZB)])

    pltpu.sync_copy(y_hbm.at[pl.ds(base, _CHUNK)], yv)
    pltpu.sync_copy(p_hbm.at[pl.ds(base, _CHUNK)], pv)
    cbase = c * _RH * _CP
    for k in range(_CHUNK // 128):
        for i in range(128 // _L):
            yi = yv[pl.ds(k * 128 + i * _L, _L)]
            pi = pv[pl.ds(k * 128 + i * _L, _L)]
            code = yi * _CP + pi - cbase
            ok = (code >= 0) & (code < _TW) & (yi != pi)
            dummy = (lax.broadcasted_iota(jnp.int32, (_L,), 0)
                     + base + k * 128 + i * _L)
            idxv[k, pl.ds(i * _L, _L)] = jnp.where(ok, code, dummy)
            valv[k, pl.ds(i * _L, _L)] = jnp.where(
                ok, jnp.full((_L,), 1.0, jnp.float32),
                jnp.zeros((_L,), jnp.float32))

    plsc.subcore_barrier()
    for k in range(_CHUNK // 128):
        pltpu.sync_copy(valv.at[k], conf_sh.at[idxv.at[k]], add=True)
    plsc.subcore_barrier()

    pltpu.sync_copy(conf_sh.at[pl.ds(s * _TSB, _TSB)], tsb)

    def col_chunk(v, _):
        def row_step(r, carry):
            m, a = carry
            for u in range(4):
                x = tsb[pl.ds((r * 4 + u) * _CP + v * _L, _L)]
                upd = x > m
                aabs = jnp.full((_L,), 0, jnp.int32) + (row0 + r * 4 + u)
                m = jnp.where(upd, x, m)
                a = jnp.where(upd, aabs, a)
            return m, a
        m, a = lax.fori_loop(
            0, _RT // 4, row_step,
            (jnp.full((_L,), -1.0, jnp.float32), jnp.zeros((_L,), jnp.int32)))
        maxb[pl.ds(v * _L, _L)] = m
        argb[pl.ds(v * _L, _L)] = a
        return 0
    lax.fori_loop(0, _CP // _L, col_chunk, 0)

    pltpu.sync_copy(maxb, maxp_sh.at[pl.ds(s * _CP, _CP)])
    pltpu.sync_copy(argb, argp_sh.at[pl.ds(s * _CP, _CP)])
    plsc.subcore_barrier()

    pltpu.sync_copy(maxp_sh, mpv)
    pltpu.sync_copy(argp_sh, apv)
    for q in range(_CC // _L):
        def comb_step(t, carry):
            m, a = carry
            x = mpv[pl.ds(t * _CP + s * _CC + q * _L, _L)]
            xa = apv[pl.ds(t * _CP + s * _CC + q * _L, _L)]
            upd = x > m
            return jnp.where(upd, x, m), jnp.where(upd, xa, a)
        m0 = mpv[pl.ds(s * _CC + q * _L, _L)]
        a0 = apv[pl.ds(s * _CC + q * _L, _L)]
        m, a = lax.fori_loop(1, _NS, comb_step, (m0, a0))
        fm[pl.ds(q * _L, _L)] = m
        fa[pl.ds(q * _L, _L)] = a

    pltpu.sync_copy(fm, entry_hbm.at[c, pl.ds(s * _CC, _CC)])
    pltpu.sync_copy(fa, argr_hbm.at[c, pl.ds(s * _CC, _CC)])


def _histogram(y, preds):
    mesh = plsc.VectorSubcoreMesh(core_axis_name="c", subcore_axis_name="s",
                                  num_cores=_NC, num_subcores=_NS)
    f = pl.kernel(
        _hist_body,
        out_type=[jax.ShapeDtypeStruct((_NC, _CP), jnp.float32),
                  jax.ShapeDtypeStruct((_NC, _CP), jnp.int32)],
        mesh=mesh,
        scratch_types=[
            pltpu.VMEM((_CHUNK,), jnp.int32),
            pltpu.VMEM((_CHUNK,), jnp.int32),
            pltpu.VMEM((_CHUNK // 128, 128), jnp.int32),
            pltpu.VMEM((_CHUNK // 128, 128), jnp.float32),
            pltpu.VMEM((_ZB,), jnp.float32),
            pltpu.VMEM((_TSB,), jnp.float32),
            pltpu.VMEM((_CP,), jnp.float32),
            pltpu.VMEM((_CP,), jnp.int32),
            pltpu.VMEM((_NS * _CP,), jnp.float32),
            pltpu.VMEM((_NS * _CP,), jnp.int32),
            pltpu.VMEM((_CC,), jnp.float32),
            pltpu.VMEM((_CC,), jnp.int32),
            pltpu.VMEM_SHARED((_TW,), jnp.float32),
            pltpu.VMEM_SHARED((_NS * _CP,), jnp.float32),
            pltpu.VMEM_SHARED((_NS * _CP,), jnp.int32),
        ],
    )
    return f(y, preds)


def _post_body(e_ref, a_ref, et_ref, at_ref, pairs_ref):
    e = e_ref[...]
    a = a_ref[...]
    e0, e1 = e[0:1, :], e[1:2, :]
    a0, a1 = a[0:1, :], a[1:2, :]
    take1 = e1 > e0
    entry = jnp.where(take1, e1, e0)
    idx = jnp.where(take1, a1, a0)
    et = et_ref[...]
    at = at_ref[...]
    e0c, e1c = et[:, 0:1], et[:, 1:2]
    a0c, a1c = at[:, 0:1], at[:, 1:2]
    take1c = e1c > e0c
    entry_c = jnp.where(take1c, e1c, e0c)
    idx_c = jnp.where(take1c, a1c, a0c)

    cio = lax.broadcasted_iota(jnp.int32, (_C, _CP), 0)
    pio = lax.broadcasted_iota(jnp.int32, (_C, _CP), 1)
    m = ((idx == cio) & (pio < _C)).astype(jnp.float32)
    counts = jnp.sum(m, axis=1, keepdims=True)
    totals = jnp.sum(m * entry, axis=1, keepdims=True)
    cioP = lax.broadcasted_iota(jnp.int32, (_CP, _C), 1)
    pioP = lax.broadcasted_iota(jnp.int32, (_CP, _C), 0)
    mP = ((idx_c == cioP) & (pioP < _C)).astype(jnp.float32)
    counts_r = jnp.sum(mP, axis=0, keepdims=True)
    totals_r = jnp.sum(mP * entry_c, axis=0, keepdims=True)

    cidx_c = lax.broadcasted_iota(jnp.int32, (_C, 1), 0)
    t_c = jnp.where(counts > 0.0, totals, -1.0)
    k2_c = (jnp.where(counts > 0.0, counts, -1.0) * 1024.0
            + (1023 - cidx_c).astype(jnp.float32))
    cidx_r = lax.broadcasted_iota(jnp.int32, (1, _C), 1)
    t_r = jnp.where(counts_r > 0.0, totals_r, -1.0)
    k2_r = (jnp.where(counts_r > 0.0, counts_r, -1.0) * 1024.0
            + (1023 - cidx_r).astype(jnp.float32))

    dom = (t_c > t_r) | ((t_c == t_r) & (k2_c > k2_r))
    rank = jnp.sum(dom.astype(jnp.float32), axis=0, keepdims=True)

    kio2 = lax.broadcasted_iota(jnp.int32, (_K, _C), 0)
    cio2 = lax.broadcasted_iota(jnp.int32, (_K, _C), 1)
    r = (kio2.astype(jnp.float32) == rank).astype(jnp.float32)

    ranked = jnp.sum(r * cio2.astype(jnp.float32), axis=1, keepdims=True)

    hit = r[:, :_W]
    tril = (lax.broadcasted_iota(jnp.int32, (_K, _K), 1)
            <= lax.broadcasted_iota(jnp.int32, (_K, _K), 0)).astype(jnp.float32)
    prefix = jnp.dot(tril, hit, preferred_element_type=jnp.float32)
    alive = (prefix == 0.0).astype(jnp.float32)
    incl = (lax.broadcasted_iota(jnp.int32, (_W, _W), 0)
            <= lax.broadcasted_iota(jnp.int32, (_W, _W), 1)).astype(jnp.float32)
    rank = jnp.dot(alive, incl, preferred_element_type=jnp.float32)

    kio = lax.broadcasted_iota(jnp.int32, (_NP, _K), 0)
    iio = lax.broadcasted_iota(jnp.int32, (_NP, _K), 1)
    start = (_K - 1) * iio - (iio * (iio - 1)) // 2
    g1 = ((kio >= start) & (kio < start + (_K - 1) - iio)).astype(jnp.float32)
    j1 = jnp.sum(g1 * (kio - start).astype(jnp.float32), axis=1,
                 keepdims=True) + 1.0

    first = jnp.dot(g1, ranked, preferred_element_type=jnp.float32)
    ai = jnp.dot(g1, alive, preferred_element_type=jnp.float32)
    ri = jnp.dot(g1, rank, preferred_element_type=jnp.float32)
    vio = lax.broadcasted_iota(jnp.int32, (_NP, _W), 1).astype(jnp.float32)
    second = jnp.sum(vio * ai * (ri == j1).astype(jnp.float32), axis=1,
                     keepdims=True)
    pairs_ref[...] = jnp.concatenate([first, second], axis=1).astype(jnp.int32)


def _post(entry2, argr2):
    return pl.pallas_call(
        _post_body,
        in_specs=[pl.BlockSpec((_NC, _CP), lambda: (0, 0)),
                  pl.BlockSpec((_NC, _CP), lambda: (0, 0)),
                  pl.BlockSpec((_CP, _NC), lambda: (0, 0)),
                  pl.BlockSpec((_CP, _NC), lambda: (0, 0))],
        out_specs=pl.BlockSpec((_NP, 2), lambda: (0, 0)),
        out_shape=jax.ShapeDtypeStruct((_NP, 2), jnp.int32),
    )(entry2, argr2, entry2.T, argr2.T)


def kernel(y, scores):
    preds = _row_argmax(scores)
    entry2, argr2 = _histogram(y, preds)
    return _post(entry2, argr2)
